# baseline (speedup 1.0000x reference)
import functools

import jax
import jax.numpy as jnp
from jax import lax
from jax.experimental import pallas as pl
from jax.experimental.pallas import tpu as pltpu
from jax.experimental.pallas import tpu_sc as plsc

N = 10000
E = 320000
D = 128
NC = 2
NS = 16
TILES = NC * NS
EPC = 96
NCH0 = 204
NCH1 = 6
EPAD = NS * (NCH0 + NCH1) * EPC
RPT = 632
NPAD = NS * RPT
TRASH = N + 64
CR = 80
NBUF = 3
NIDX = 6
UNROLL = 6

_MESH = dict(
    mesh=plsc.VectorSubcoreMesh(core_axis_name="c", subcore_axis_name="s"),
    compiler_params=pltpu.CompilerParams(needs_layout_passes=False),
)


def _make_agg(count):
    out_type = [jax.ShapeDtypeStruct((NC, NPAD, D), jnp.float32)]
    scratch = (
        [pltpu.VMEM_SHARED((NPAD, D), jnp.float32)]
        + [pltpu.VMEM((EPC,), jnp.int32) for _ in range(2 * NIDX)]
        + [pltpu.VMEM((EPC, D), jnp.float32) for _ in range(NBUF)]
        + [pltpu.SemaphoreType.DMA for _ in range(2 * NBUF + NIDX)]
    )
    if count:
        out_type.append(jax.ShapeDtypeStruct((TILES * CR * D,), jnp.float32))
        scratch.append(pltpu.VMEM((CR * D,), jnp.float32))

    @functools.partial(pl.kernel, out_type=out_type, scratch_types=scratch,
                       **_MESH)
    def agg(table, src2, dst2, out, *rest):
        if count:
            out_cnt, accum = rest[0], rest[1]
            cnt_v = rest[-1]
            rest = rest[2:-1]
        else:
            accum = rest[0]
            rest = rest[1:]
        _agg_body(count, table, src2, dst2, out,
                  out_cnt if count else None, accum,
                  cnt_v if count else None, rest)

    return agg


def _agg_body(count, table, src2, dst2, out, out_cnt, accum, cnt_v, rest):
    sidx = rest[:NIDX]
    didx = rest[NIDX:2 * NIDX]
    rows = rest[2 * NIDX:2 * NIDX + NBUF]
    sg = rest[2 * NIDX + NBUF:2 * NIDX + 2 * NBUF]
    ss = rest[2 * NIDX + 2 * NBUF:2 * NIDX + 3 * NBUF]
    si = rest[2 * NIDX + 3 * NBUF:]
    c = lax.axis_index("c")
    s = lax.axis_index("s")
    base = jnp.where(c == 0, s * NCH0, NS * NCH0 + s * NCH1)
    nch = jnp.where(c == 0, NCH0, NCH1)
    z16 = jnp.zeros((16,), jnp.float32)

    def zrow(r, carry):
        for kk in range(D // 16):
            rows[0][r, pl.ds(kk * 16, 16)] = z16
        return carry

    lax.fori_loop(0, EPC, zrow, 0)
    for j in range(RPT // EPC):
        pltpu.sync_copy(rows[0], accum.at[pl.ds(s * RPT + j * EPC, EPC)])
    rem = RPT - (RPT // EPC) * EPC
    pltpu.sync_copy(rows[0].at[pl.ds(0, rem)],
                    accum.at[pl.ds(s * RPT + RPT - rem, rem)])
    if count:
        def zcnt(r, carry):
            cnt_v[pl.ds(r * 16, 16)] = z16
            return carry

        lax.fori_loop(0, CR * D // 16, zcnt, 0)
    one16 = jnp.ones((16,), jnp.float32)
    plsc.subcore_barrier()

    def load_idx(i, q):
        pltpu.async_copy(src2.at[base + i], sidx[q], si[q])
        pltpu.async_copy(dst2.at[base + i], didx[q], si[q])

    def wait_idx(q):
        pltpu.make_async_copy(src2.at[base], sidx[q], si[q]).wait()
        pltpu.make_async_copy(dst2.at[base], didx[q], si[q]).wait()

    def gather(q, b):
        pltpu.async_copy(table.at[sidx[q]], rows[b], sg[b])

    def wait_gather(q, b):
        pltpu.make_async_copy(table.at[sidx[q]], rows[b], sg[b]).wait()

    def scatter(q, b):
        pltpu.async_copy(rows[b], accum.at[didx[q]], ss[b], add=True)

    def wait_scatter(q, b):
        pltpu.make_async_copy(rows[b], accum.at[didx[q]], ss[b]).wait()

    for j in range(4):
        load_idx(j, j)
    wait_idx(0)
    wait_idx(1)
    gather(0, 0)
    gather(1, 1)

    def group(g, carry):
        for u in range(UNROLL):
            i = g * UNROLL + u
            b = u % NBUF

            @pl.when(i >= 1)
            def _():
                wait_scatter((u + 5) % NIDX, (u + 2) % NBUF)

            @pl.when(i + 2 < nch)
            def _():
                wait_idx((u + 2) % NIDX)
                gather((u + 2) % NIDX, (u + 2) % NBUF)

            @pl.when(i + 4 < nch)
            def _():
                load_idx(i + 4, (u + 4) % NIDX)

            if count:
                for k2 in range(EPC // 16):
                    d16 = didx[u][pl.ds(k2 * 16, 16)]
                    plsc.addupdate_scatter(cnt_v, [d16], one16)
            wait_gather(u, b)
            scatter(u, b)
        return carry

    lax.fori_loop(0, nch // UNROLL, group, 0)
    wait_scatter(5, 2)
    plsc.subcore_barrier()
    pltpu.sync_copy(accum.at[pl.ds(s * RPT, RPT)],
                    out.at[c, pl.ds(s * RPT, RPT)])
    if count:
        w = c * NS + s
        pltpu.sync_copy(cnt_v, out_cnt.at[pl.ds(w * CR * D, CR * D)])


_agg_cnt = _make_agg(True)
_agg = _make_agg(False)


def _layer_norm(y, w, b):
    mu = jnp.mean(y, axis=-1, keepdims=True)
    var = jnp.mean((y - mu) ** 2, axis=-1, keepdims=True)
    return (y - mu) * lax.rsqrt(var + 1e-5) * w + b


def _rec_col(cnt_ref):
    g = jnp.sum(cnt_ref[...], axis=0)
    rg = 1.0 / jnp.maximum(g, 1.0)
    r8 = lax.broadcasted_iota(jnp.int32, (_B, _B // D), 0) // D
    j8 = lax.broadcasted_iota(jnp.int32, (_B, _B // D), 1)
    m = (r8 == j8).astype(jnp.float32)
    a = jnp.dot(m, rg, preferred_element_type=jnp.float32)
    rl = lax.broadcasted_iota(jnp.int32, (_B, D), 0) % D
    ll = lax.broadcasted_iota(jnp.int32, (_B, D), 1)
    sel = (rl == ll).astype(jnp.float32)
    return jnp.sum(a * sel, axis=1, keepdims=True)


def _tc1_body(p0, p1, cnt, x, wl, bl, wr, lnw, lnb, h_ref):
    mean = (p0[...] + p1[...]) * _rec_col(cnt)
    y = jnp.dot(mean, wl[...], preferred_element_type=jnp.float32) + bl[...]
    y = y + jnp.dot(x[...], wr[...], preferred_element_type=jnp.float32)
    y = _layer_norm(y, lnw[...], lnb[...])
    h_ref[...] = jnp.maximum(y, 0.0)


def _tc2_body(q0, q1, cnt, h, x, wl, bl, wr, lnw, lnb, o_ref):
    mean = (q0[...] + q1[...]) * _rec_col(cnt)
    y = jnp.dot(mean, wl[...], preferred_element_type=jnp.float32) + bl[...]
    y = y + jnp.dot(h[...], wr[...], preferred_element_type=jnp.float32) + x[...]
    y = _layer_norm(y, lnw[...], lnb[...])
    o_ref[...] = jnp.maximum(y, 0.0)


_B = 1024
_row = pl.BlockSpec((_B, D), lambda i: (i, 0))
_crow = pl.BlockSpec((TILES, _B // D, D), lambda i: (0, i, 0))
_full = pl.BlockSpec((D, D), lambda i: (0, 0))
_vec = pl.BlockSpec((D,), lambda i: (0,))


def _tc1(p0, p1, cnt, x, wl, bl, wr, lnw, lnb):
    return pl.pallas_call(
        _tc1_body,
        grid=(pl.cdiv(N, _B),),
        in_specs=[_row, _row, _crow, _row, _full, _vec, _full, _vec, _vec],
        out_specs=_row,
        out_shape=jax.ShapeDtypeStruct((N, D), jnp.float32),
    )(p0, p1, cnt, x, wl, bl, wr, lnw, lnb)


def _tc2(q0, q1, cnt, h, x, wl, bl, wr, lnw, lnb):
    return pl.pallas_call(
        _tc2_body,
        grid=(pl.cdiv(N, _B),),
        in_specs=[_row, _row, _crow, _row, _row, _full, _vec, _full, _vec, _vec],
        out_specs=_row,
        out_shape=jax.ShapeDtypeStruct((N, D), jnp.float32),
    )(q0, q1, cnt, h, x, wl, bl, wr, lnw, lnb)


def kernel(x, edge_index, Wl0, bl0, Wr0, Wl1, bl1, Wr1, ln0_w, ln0_b, ln1_w, ln1_b):
    src = edge_index[0]
    dst = edge_index[1]
    pad = EPAD - E
    nrows = EPAD // EPC
    src2 = jnp.concatenate([src, jnp.zeros((pad,), jnp.int32)]).reshape(
        nrows, EPC)
    dst2 = jnp.concatenate([dst, jnp.full((pad,), TRASH, jnp.int32)]).reshape(
        nrows, EPC)

    parts1, cnt_flat = _agg_cnt(x, src2, dst2)
    cnt_t = cnt_flat.reshape(TILES, CR, D)
    h = _tc1(parts1[0, :N], parts1[1, :N], cnt_t, x,
             Wl0.T, bl0, Wr0.T, ln0_w, ln0_b)
    parts2 = _agg(h, src2, dst2)[0]
    out = _tc2(parts2[0, :N], parts2[1, :N], cnt_t, h, x,
               Wl1.T, bl1, Wr1.T, ln1_w, ln1_b)
    return out


# speedup vs baseline: 1.1151x; 1.1151x over previous
import functools

import jax
import jax.numpy as jnp
from jax import lax
from jax.experimental import pallas as pl
from jax.experimental.pallas import tpu as pltpu
from jax.experimental.pallas import tpu_sc as plsc

N = 10000
E = 320000
D = 128
NC = 2
NS = 16
TILES = NC * NS
EPC = 96
NCH0 = 198
NCH1 = 12
EPAD = NS * (NCH0 + NCH1) * EPC
RPT = 632
NPAD = NS * RPT
TRASH = N + 64
CR = 80
NBUF = 3
NIDX = 6
UNROLL = 6

_MESH = dict(
    mesh=plsc.VectorSubcoreMesh(core_axis_name="c", subcore_axis_name="s"),
    compiler_params=pltpu.CompilerParams(needs_layout_passes=False),
)


def _make_agg(count):
    out_type = [jax.ShapeDtypeStruct((NC, NPAD, D), jnp.float32)]
    scratch = (
        [pltpu.VMEM_SHARED((NPAD, D), jnp.float32)]
        + [pltpu.VMEM((EPC,), jnp.int32) for _ in range(2 * NIDX)]
        + [pltpu.VMEM((EPC, D), jnp.float32) for _ in range(NBUF)]
        + [pltpu.SemaphoreType.DMA for _ in range(2 * NBUF + NIDX)]
    )
    if count:
        out_type.append(jax.ShapeDtypeStruct((TILES * CR * D,), jnp.float32))
        scratch.append(pltpu.VMEM((CR * D,), jnp.float32))

    @functools.partial(pl.kernel, out_type=out_type, scratch_types=scratch,
                       **_MESH)
    def agg(table, src2, dst2, out, *rest):
        if count:
            out_cnt, accum = rest[0], rest[1]
            cnt_v = rest[-1]
            rest = rest[2:-1]
        else:
            accum = rest[0]
            rest = rest[1:]
        _agg_body(count, table, src2, dst2, out,
                  out_cnt if count else None, accum,
                  cnt_v if count else None, rest)

    return agg


def _agg_body(count, table, src2, dst2, out, out_cnt, accum, cnt_v, rest):
    sidx = rest[:NIDX]
    didx = rest[NIDX:2 * NIDX]
    rows = rest[2 * NIDX:2 * NIDX + NBUF]
    sg = rest[2 * NIDX + NBUF:2 * NIDX + 2 * NBUF]
    ss = rest[2 * NIDX + 2 * NBUF:2 * NIDX + 3 * NBUF]
    si = rest[2 * NIDX + 3 * NBUF:]
    c = lax.axis_index("c")
    s = lax.axis_index("s")
    base = jnp.where(c == 0, s * NCH0, NS * NCH0 + s * NCH1)
    nch = jnp.where(c == 0, NCH0, NCH1)
    z16 = jnp.zeros((16,), jnp.float32)

    def zrow(r, carry):
        for kk in range(D // 16):
            rows[0][r, pl.ds(kk * 16, 16)] = z16
        return carry

    lax.fori_loop(0, EPC, zrow, 0)
    for j in range(RPT // EPC):
        pltpu.sync_copy(rows[0], accum.at[pl.ds(s * RPT + j * EPC, EPC)])
    rem = RPT - (RPT // EPC) * EPC
    pltpu.sync_copy(rows[0].at[pl.ds(0, rem)],
                    accum.at[pl.ds(s * RPT + RPT - rem, rem)])
    if count:
        def zcnt(r, carry):
            cnt_v[pl.ds(r * 16, 16)] = z16
            return carry

        lax.fori_loop(0, CR * D // 16, zcnt, 0)
    one16 = jnp.ones((16,), jnp.float32)
    plsc.subcore_barrier()

    def load_idx(i, q):
        pltpu.async_copy(src2.at[base + i], sidx[q], si[q])
        pltpu.async_copy(dst2.at[base + i], didx[q], si[q])

    def wait_idx(q):
        pltpu.make_async_copy(src2.at[base], sidx[q], si[q]).wait()
        pltpu.make_async_copy(dst2.at[base], didx[q], si[q]).wait()

    def gather(q, b):
        pltpu.async_copy(table.at[sidx[q]], rows[b], sg[b])

    def wait_gather(q, b):
        pltpu.make_async_copy(table.at[sidx[q]], rows[b], sg[b]).wait()

    def scatter(q, b):
        pltpu.async_copy(rows[b], accum.at[didx[q]], ss[b], add=True)

    def wait_scatter(q, b):
        pltpu.make_async_copy(rows[b], accum.at[didx[q]], ss[b]).wait()

    for j in range(4):
        load_idx(j, j)
    wait_idx(0)
    wait_idx(1)
    gather(0, 0)
    gather(1, 1)

    def group(g, carry):
        for u in range(UNROLL):
            i = g * UNROLL + u
            b = u % NBUF

            @pl.when(i >= 1)
            def _():
                wait_scatter((u + 5) % NIDX, (u + 2) % NBUF)

            @pl.when(i + 2 < nch)
            def _():
                wait_idx((u + 2) % NIDX)
                gather((u + 2) % NIDX, (u + 2) % NBUF)

            @pl.when(i + 4 < nch)
            def _():
                load_idx(i + 4, (u + 4) % NIDX)

            if count:
                for k2 in range(EPC // 16):
                    d16 = didx[u][pl.ds(k2 * 16, 16)]
                    plsc.addupdate_scatter(cnt_v, [d16], one16)
            wait_gather(u, b)
            scatter(u, b)
        return carry

    lax.fori_loop(0, nch // UNROLL, group, 0)
    wait_scatter(5, 2)
    plsc.subcore_barrier()
    pltpu.sync_copy(accum.at[pl.ds(s * RPT, RPT)],
                    out.at[c, pl.ds(s * RPT, RPT)])
    if count:
        w = c * NS + s
        pltpu.sync_copy(cnt_v, out_cnt.at[pl.ds(w * CR * D, CR * D)])


_agg_cnt = _make_agg(True)
_agg = _make_agg(False)


def _layer_norm(y, w, b):
    mu = jnp.mean(y, axis=-1, keepdims=True)
    var = jnp.mean((y - mu) ** 2, axis=-1, keepdims=True)
    return (y - mu) * lax.rsqrt(var + 1e-5) * w + b


def _rec_col(cnt_ref):
    g = jnp.sum(cnt_ref[...], axis=0)
    rg = 1.0 / jnp.maximum(g, 1.0)
    r8 = lax.broadcasted_iota(jnp.int32, (_B, _B // D), 0) // D
    j8 = lax.broadcasted_iota(jnp.int32, (_B, _B // D), 1)
    m = (r8 == j8).astype(jnp.float32)
    a = jnp.dot(m, rg, preferred_element_type=jnp.float32)
    rl = lax.broadcasted_iota(jnp.int32, (_B, D), 0) % D
    ll = lax.broadcasted_iota(jnp.int32, (_B, D), 1)
    sel = (rl == ll).astype(jnp.float32)
    return jnp.sum(a * sel, axis=1, keepdims=True)


def _tc1_body(p0, p1, cnt, x, wl, bl, wr, lnw, lnb, h_ref):
    mean = (p0[...] + p1[...]) * _rec_col(cnt)
    y = jnp.dot(mean, wl[...], preferred_element_type=jnp.float32) + bl[...]
    y = y + jnp.dot(x[...], wr[...], preferred_element_type=jnp.float32)
    y = _layer_norm(y, lnw[...], lnb[...])
    h_ref[...] = jnp.maximum(y, 0.0)


def _tc2_body(q0, q1, cnt, h, x, wl, bl, wr, lnw, lnb, o_ref):
    mean = (q0[...] + q1[...]) * _rec_col(cnt)
    y = jnp.dot(mean, wl[...], preferred_element_type=jnp.float32) + bl[...]
    y = y + jnp.dot(h[...], wr[...], preferred_element_type=jnp.float32) + x[...]
    y = _layer_norm(y, lnw[...], lnb[...])
    o_ref[...] = jnp.maximum(y, 0.0)


_B = 1024
_row = pl.BlockSpec((_B, D), lambda i: (i, 0))
_crow = pl.BlockSpec((TILES, _B // D, D), lambda i: (0, i, 0))
_full = pl.BlockSpec((D, D), lambda i: (0, 0))
_vec = pl.BlockSpec((D,), lambda i: (0,))


def _tc1(p0, p1, cnt, x, wl, bl, wr, lnw, lnb):
    return pl.pallas_call(
        _tc1_body,
        grid=(pl.cdiv(N, _B),),
        in_specs=[_row, _row, _crow, _row, _full, _vec, _full, _vec, _vec],
        out_specs=_row,
        out_shape=jax.ShapeDtypeStruct((N, D), jnp.float32),
    )(p0, p1, cnt, x, wl, bl, wr, lnw, lnb)


def _tc2(q0, q1, cnt, h, x, wl, bl, wr, lnw, lnb):
    return pl.pallas_call(
        _tc2_body,
        grid=(pl.cdiv(N, _B),),
        in_specs=[_row, _row, _crow, _row, _row, _full, _vec, _full, _vec, _vec],
        out_specs=_row,
        out_shape=jax.ShapeDtypeStruct((N, D), jnp.float32),
    )(q0, q1, cnt, h, x, wl, bl, wr, lnw, lnb)


def kernel(x, edge_index, Wl0, bl0, Wr0, Wl1, bl1, Wr1, ln0_w, ln0_b, ln1_w, ln1_b):
    src = edge_index[0]
    dst = edge_index[1]
    pad = EPAD - E
    nrows = EPAD // EPC
    src2 = jnp.concatenate([src, jnp.zeros((pad,), jnp.int32)]).reshape(
        nrows, EPC)
    dst2 = jnp.concatenate([dst, jnp.full((pad,), TRASH, jnp.int32)]).reshape(
        nrows, EPC)

    parts1, cnt_flat = _agg_cnt(x, src2, dst2)
    cnt_t = cnt_flat.reshape(TILES, CR, D)
    h = _tc1(parts1[0, :N], parts1[1, :N], cnt_t, x,
             Wl0.T, bl0, Wr0.T, ln0_w, ln0_b)
    parts2 = _agg(h, src2, dst2)[0]
    out = _tc2(parts2[0, :N], parts2[1, :N], cnt_t, h, x,
               Wl1.T, bl1, Wr1.T, ln1_w, ln1_b)
    return out
